# baseline (speedup 1.0000x reference)
import functools

import jax
import jax.numpy as jnp
from jax import lax
from jax.experimental import pallas as pl
from jax.experimental.pallas import tpu as pltpu
from jax.experimental.pallas import tpu_sc as plsc

DIM = 512
CB = 256
NCB = 16
DPC = 32
TBLK = 1024
NW = 32
SCROWS = 64


def _vq_kernel(x_ref, a_ref, b_ref, out_ref, nrm_ref, acc_ref):
    i = pl.program_id(0)

    @pl.when(i == 0)
    def _init():
        acc_ref[0] = 0.0
        for c in range(NCB):
            tt = a_ref[c * DPC:(c + 1) * DPC, CB:].astype(jnp.float32)
            nrm_ref[c:c + 1, :] = 0.25 * jnp.sum(tt * tt, axis=0,
                                                 keepdims=True)

    xbf = x_ref[...].astype(jnp.bfloat16)
    vaccs = [jnp.zeros((TBLK, CB), jnp.float32) for _ in range(4)]
    for c in range(NCB):
        ac = a_ref[c * DPC:(c + 1) * DPC, :]
        prod = jnp.dot(xbf[:, c * DPC:(c + 1) * DPC], ac,
                       preferred_element_type=jnp.float32)
        logits = prod[:, :CB] + b_ref[c:c + 1, :]
        m = jnp.max(logits, axis=1, keepdims=True)
        fval = prod[:, CB:] + nrm_ref[c:c + 1, :]
        vaccs[c % 4] += jnp.where(logits == m, fval, 0.0)
    vacc = (vaccs[0] + vaccs[1]) + (vaccs[2] + vaccs[3])
    acc_ref[0] += jnp.sum(vacc)

    @pl.when(i == pl.num_programs(0) - 1)
    def _fin():
        out_ref[...] = jnp.full((1, 1), acc_ref[0], dtype=jnp.float32)


def _make_sc_sumsq(n_tokens):
    rows_per_w = n_tokens // NW
    nchunk = rows_per_w // SCROWS
    groups = SCROWS * DIM // 16
    mesh = plsc.VectorSubcoreMesh(core_axis_name="c", subcore_axis_name="s")

    @functools.partial(
        pl.kernel, mesh=mesh,
        out_type=jax.ShapeDtypeStruct((NW, 16), jnp.float32),
        scratch_types=[
            pltpu.VMEM((SCROWS, DIM), jnp.float32),
            pltpu.VMEM((SCROWS, DIM), jnp.float32),
            pltpu.VMEM((16,), jnp.float32),
            pltpu.SemaphoreType.DMA,
            pltpu.SemaphoreType.DMA,
        ],
    )
    def sc_sumsq(x_hbm, out_hbm, buf0, buf1, acc_v, sem0, sem1):
        wid = lax.axis_index("s") * 2 + lax.axis_index("c")
        base = wid * rows_per_w
        acc_v[...] = jnp.zeros((16,), jnp.float32)
        bufs = (buf0, buf1)
        sems = (sem0, sem1)
        cps = [None, None]
        cps[0] = pltpu.async_copy(
            x_hbm.at[pl.ds(base, SCROWS)], buf0, sem0)
        for j in range(nchunk):
            nxt = (j + 1) % 2
            cur = j % 2
            if j + 1 < nchunk:
                cps[nxt] = pltpu.async_copy(
                    x_hbm.at[pl.ds(base + (j + 1) * SCROWS, SCROWS)],
                    bufs[nxt], sems[nxt])
            cps[cur].wait()
            buf = bufs[cur]

            @pl.loop(0, SCROWS)
            def _row(r):
                parts = [jnp.zeros((16,), jnp.float32) for _ in range(4)]
                for l in range(DIM // 16):
                    v = buf[r, pl.ds(l * 16, 16)]
                    parts[l % 4] = parts[l % 4] + v * v
                acc_v[...] = acc_v[...] + ((parts[0] + parts[1]) +
                                           (parts[2] + parts[3]))

        pltpu.sync_copy(acc_v, out_hbm.at[wid])

    return sc_sumsq


def kernel(x, W, b, to_output, mask):
    del mask
    n_tokens = x.shape[0]

    w4 = W.reshape(NCB, CB, NCB, DPC)
    t4 = to_output.reshape(NCB, CB, NCB, DPC)
    diag = jnp.arange(NCB)
    wblk = w4[diag, :, diag, :]
    tblk = t4[diag, :, diag, :]
    a = jnp.concatenate(
        [jnp.transpose(wblk, (0, 2, 1)).reshape(NCB * DPC, CB),
         -2.0 * jnp.transpose(tblk, (0, 2, 1)).reshape(NCB * DPC, CB)],
        axis=1).astype(jnp.bfloat16)
    b2 = b.reshape(NCB, CB)

    grid = n_tokens // TBLK
    num = pl.pallas_call(
        _vq_kernel,
        grid=(grid,),
        in_specs=[
            pl.BlockSpec((TBLK, DIM), lambda i: (i, 0)),
            pl.BlockSpec((NCB * DPC, 2 * CB), lambda i: (0, 0)),
            pl.BlockSpec((NCB, CB), lambda i: (0, 0)),
        ],
        out_specs=pl.BlockSpec((1, 1), lambda i: (0, 0)),
        out_shape=jax.ShapeDtypeStruct((1, 1), jnp.float32),
        scratch_shapes=[pltpu.VMEM((NCB, CB), jnp.float32),
                        pltpu.SMEM((1,), jnp.float32)],
    )(x, a, b2)

    sxp = _make_sc_sumsq(n_tokens)(x)
    sx = jnp.sum(sxp)
    return (num.reshape(()) + sx) / (sx + 1e-20)


# speedup vs baseline: 1.0010x; 1.0010x over previous
import functools

import jax
import jax.numpy as jnp
from jax import lax
from jax.experimental import pallas as pl
from jax.experimental.pallas import tpu as pltpu
from jax.experimental.pallas import tpu_sc as plsc

DIM = 512
CB = 256
NCB = 16
DPC = 32
TBLK = 2048
NW = 32
SCROWS = 64


def _vq_kernel(x_ref, a_ref, b_ref, out_ref, nrm_ref, acc_ref):
    i = pl.program_id(0)

    @pl.when(i == 0)
    def _init():
        acc_ref[0] = 0.0
        for c in range(NCB):
            tt = a_ref[c * DPC:(c + 1) * DPC, CB:].astype(jnp.float32)
            nrm_ref[c:c + 1, :] = 0.25 * jnp.sum(tt * tt, axis=0,
                                                 keepdims=True)

    xbf = x_ref[...].astype(jnp.bfloat16)
    vaccs = [jnp.zeros((TBLK, CB), jnp.float32) for _ in range(4)]
    for c in range(NCB):
        ac = a_ref[c * DPC:(c + 1) * DPC, :]
        prod = jnp.dot(xbf[:, c * DPC:(c + 1) * DPC], ac,
                       preferred_element_type=jnp.float32)
        logits = prod[:, :CB] + b_ref[c:c + 1, :]
        m = jnp.max(logits, axis=1, keepdims=True)
        fval = prod[:, CB:] + nrm_ref[c:c + 1, :]
        vaccs[c % 4] += jnp.where(logits == m, fval, 0.0)
    vacc = (vaccs[0] + vaccs[1]) + (vaccs[2] + vaccs[3])
    acc_ref[0] += jnp.sum(vacc)

    @pl.when(i == pl.num_programs(0) - 1)
    def _fin():
        out_ref[...] = jnp.full((1, 1), acc_ref[0], dtype=jnp.float32)


def _make_sc_sumsq(n_tokens):
    rows_per_w = n_tokens // NW
    nchunk = rows_per_w // SCROWS
    groups = SCROWS * DIM // 16
    mesh = plsc.VectorSubcoreMesh(core_axis_name="c", subcore_axis_name="s")

    @functools.partial(
        pl.kernel, mesh=mesh,
        out_type=jax.ShapeDtypeStruct((NW, 16), jnp.float32),
        scratch_types=[
            pltpu.VMEM((SCROWS, DIM), jnp.float32),
            pltpu.VMEM((SCROWS, DIM), jnp.float32),
            pltpu.VMEM((16,), jnp.float32),
            pltpu.SemaphoreType.DMA,
            pltpu.SemaphoreType.DMA,
        ],
    )
    def sc_sumsq(x_hbm, out_hbm, buf0, buf1, acc_v, sem0, sem1):
        wid = lax.axis_index("s") * 2 + lax.axis_index("c")
        base = wid * rows_per_w
        acc_v[...] = jnp.zeros((16,), jnp.float32)
        bufs = (buf0, buf1)
        sems = (sem0, sem1)
        cps = [None, None]
        cps[0] = pltpu.async_copy(
            x_hbm.at[pl.ds(base, SCROWS)], buf0, sem0)
        for j in range(nchunk):
            nxt = (j + 1) % 2
            cur = j % 2
            if j + 1 < nchunk:
                cps[nxt] = pltpu.async_copy(
                    x_hbm.at[pl.ds(base + (j + 1) * SCROWS, SCROWS)],
                    bufs[nxt], sems[nxt])
            cps[cur].wait()
            buf = bufs[cur]

            @pl.loop(0, SCROWS)
            def _row(r):
                parts = [jnp.zeros((16,), jnp.float32) for _ in range(4)]
                for l in range(DIM // 16):
                    v = buf[r, pl.ds(l * 16, 16)]
                    parts[l % 4] = parts[l % 4] + v * v
                acc_v[...] = acc_v[...] + ((parts[0] + parts[1]) +
                                           (parts[2] + parts[3]))

        pltpu.sync_copy(acc_v, out_hbm.at[wid])

    return sc_sumsq


def kernel(x, W, b, to_output, mask):
    del mask
    n_tokens = x.shape[0]

    w4 = W.reshape(NCB, CB, NCB, DPC)
    t4 = to_output.reshape(NCB, CB, NCB, DPC)
    diag = jnp.arange(NCB)
    wblk = w4[diag, :, diag, :]
    tblk = t4[diag, :, diag, :]
    a = jnp.concatenate(
        [jnp.transpose(wblk, (0, 2, 1)).reshape(NCB * DPC, CB),
         -2.0 * jnp.transpose(tblk, (0, 2, 1)).reshape(NCB * DPC, CB)],
        axis=1).astype(jnp.bfloat16)
    b2 = b.reshape(NCB, CB)

    grid = n_tokens // TBLK
    num = pl.pallas_call(
        _vq_kernel,
        grid=(grid,),
        in_specs=[
            pl.BlockSpec((TBLK, DIM), lambda i: (i, 0)),
            pl.BlockSpec((NCB * DPC, 2 * CB), lambda i: (0, 0)),
            pl.BlockSpec((NCB, CB), lambda i: (0, 0)),
        ],
        out_specs=pl.BlockSpec((1, 1), lambda i: (0, 0)),
        out_shape=jax.ShapeDtypeStruct((1, 1), jnp.float32),
        scratch_shapes=[pltpu.VMEM((NCB, CB), jnp.float32),
                        pltpu.SMEM((1,), jnp.float32)],
    )(x, a, b2)

    sxp = _make_sc_sumsq(n_tokens)(x)
    sx = jnp.sum(sxp)
    return (num.reshape(()) + sx) / (sx + 1e-20)


# speedup vs baseline: 1.0742x; 1.0731x over previous
import functools

import jax
import jax.numpy as jnp
from jax import lax
from jax.experimental import pallas as pl
from jax.experimental.pallas import tpu as pltpu
from jax.experimental.pallas import tpu_sc as plsc

DIM = 512
CB = 256
NCB = 16
DPC = 32
TBLK = 2048
NW = 32
SCROWS = 64


def _vq_kernel(x_ref, a_ref, b_ref, out_ref, nrm_ref, acc_ref):
    i = pl.program_id(0)

    @pl.when(i == 0)
    def _init():
        acc_ref[0] = 0.0
        for c in range(NCB):
            tt = a_ref[c * DPC:(c + 1) * DPC, CB:].astype(jnp.float32)
            nrm_ref[c:c + 1, :] = 0.25 * jnp.sum(tt * tt, axis=0,
                                                 keepdims=True)

    xbf = x_ref[...].astype(jnp.bfloat16)
    accs = [jnp.zeros((1, CB), jnp.float32) for _ in range(4)]
    for c in range(NCB):
        ac = a_ref[c * DPC:(c + 1) * DPC, :]
        prod = jnp.dot(xbf[:, c * DPC:(c + 1) * DPC], ac,
                       preferred_element_type=jnp.float32)
        logits = prod[:, :CB] + b_ref[c:c + 1, :]
        m = jnp.max(logits, axis=1, keepdims=True)
        fval = prod[:, CB:] + nrm_ref[c:c + 1, :]
        val = jnp.where(logits == m, fval, 0.0)
        accs[c % 4] += jnp.sum(val, axis=0, keepdims=True)
    acc = (accs[0] + accs[1]) + (accs[2] + accs[3])
    acc_ref[0] += jnp.sum(acc)

    @pl.when(i == pl.num_programs(0) - 1)
    def _fin():
        out_ref[...] = jnp.full((1, 1), acc_ref[0], dtype=jnp.float32)


def _make_sc_sumsq(n_tokens):
    rows_per_w = n_tokens // NW
    nchunk = rows_per_w // SCROWS
    groups = SCROWS * DIM // 16
    mesh = plsc.VectorSubcoreMesh(core_axis_name="c", subcore_axis_name="s")

    @functools.partial(
        pl.kernel, mesh=mesh,
        out_type=jax.ShapeDtypeStruct((NW, 16), jnp.float32),
        scratch_types=[
            pltpu.VMEM((SCROWS, DIM), jnp.float32),
            pltpu.VMEM((SCROWS, DIM), jnp.float32),
            pltpu.VMEM((16,), jnp.float32),
            pltpu.SemaphoreType.DMA,
            pltpu.SemaphoreType.DMA,
        ],
    )
    def sc_sumsq(x_hbm, out_hbm, buf0, buf1, acc_v, sem0, sem1):
        wid = lax.axis_index("s") * 2 + lax.axis_index("c")
        base = wid * rows_per_w
        acc_v[...] = jnp.zeros((16,), jnp.float32)
        bufs = (buf0, buf1)
        sems = (sem0, sem1)
        cps = [None, None]
        cps[0] = pltpu.async_copy(
            x_hbm.at[pl.ds(base, SCROWS)], buf0, sem0)
        for j in range(nchunk):
            nxt = (j + 1) % 2
            cur = j % 2
            if j + 1 < nchunk:
                cps[nxt] = pltpu.async_copy(
                    x_hbm.at[pl.ds(base + (j + 1) * SCROWS, SCROWS)],
                    bufs[nxt], sems[nxt])
            cps[cur].wait()
            buf = bufs[cur]

            @pl.loop(0, SCROWS)
            def _row(r):
                parts = [jnp.zeros((16,), jnp.float32) for _ in range(4)]
                for l in range(DIM // 16):
                    v = buf[r, pl.ds(l * 16, 16)]
                    parts[l % 4] = parts[l % 4] + v * v
                acc_v[...] = acc_v[...] + ((parts[0] + parts[1]) +
                                           (parts[2] + parts[3]))

        pltpu.sync_copy(acc_v, out_hbm.at[wid])

    return sc_sumsq


def kernel(x, W, b, to_output, mask):
    del mask
    n_tokens = x.shape[0]

    w4 = W.reshape(NCB, CB, NCB, DPC)
    t4 = to_output.reshape(NCB, CB, NCB, DPC)
    diag = jnp.arange(NCB)
    wblk = w4[diag, :, diag, :]
    tblk = t4[diag, :, diag, :]
    a = jnp.concatenate(
        [jnp.transpose(wblk, (0, 2, 1)).reshape(NCB * DPC, CB),
         -2.0 * jnp.transpose(tblk, (0, 2, 1)).reshape(NCB * DPC, CB)],
        axis=1).astype(jnp.bfloat16)
    b2 = b.reshape(NCB, CB)

    grid = n_tokens // TBLK
    num = pl.pallas_call(
        _vq_kernel,
        grid=(grid,),
        in_specs=[
            pl.BlockSpec((TBLK, DIM), lambda i: (i, 0)),
            pl.BlockSpec((NCB * DPC, 2 * CB), lambda i: (0, 0)),
            pl.BlockSpec((NCB, CB), lambda i: (0, 0)),
        ],
        out_specs=pl.BlockSpec((1, 1), lambda i: (0, 0)),
        out_shape=jax.ShapeDtypeStruct((1, 1), jnp.float32),
        scratch_shapes=[pltpu.VMEM((NCB, CB), jnp.float32),
                        pltpu.SMEM((1,), jnp.float32)],
    )(x, a, b2)

    sxp = _make_sc_sumsq(n_tokens)(x)
    sx = jnp.sum(sxp)
    return (num.reshape(()) + sx) / (sx + 1e-20)


# speedup vs baseline: 1.0753x; 1.0011x over previous
import functools

import jax
import jax.numpy as jnp
from jax import lax
from jax.experimental import pallas as pl
from jax.experimental.pallas import tpu as pltpu
from jax.experimental.pallas import tpu_sc as plsc

DIM = 512
CB = 256
NCB = 16
DPC = 32
TBLK = 2048
NW = 32
SCROWS = 64


def _vq_kernel(x_ref, a_ref, b_ref, out_ref, nrm_ref, acc_ref):
    i = pl.program_id(0)

    @pl.when(i == 0)
    def _init():
        acc_ref[0] = 0.0
        for c in range(NCB):
            tt = a_ref[c * DPC:(c + 1) * DPC, CB:].astype(jnp.float32)
            nrm_ref[c:c + 1, :] = 0.25 * jnp.sum(tt * tt, axis=0,
                                                 keepdims=True)

    xbf = x_ref[...].astype(jnp.bfloat16)
    accs = [jnp.zeros((1, CB), jnp.float32) for _ in range(4)]
    for c in range(NCB):
        ac = a_ref[c * DPC:(c + 1) * DPC, :]
        prod = jnp.dot(xbf[:, c * DPC:(c + 1) * DPC], ac,
                       preferred_element_type=jnp.float32)
        logits = prod[:, :CB] + b_ref[c:c + 1, :]
        m = jnp.max(logits, axis=1, keepdims=True)
        fval = prod[:, CB:] + nrm_ref[c:c + 1, :]
        val = jnp.where(logits == m, fval, 0.0)
        accs[c % 4] += jnp.sum(val, axis=0, keepdims=True)
    acc = (accs[0] + accs[1]) + (accs[2] + accs[3])
    acc_ref[0] += jnp.sum(acc)

    @pl.when(i == pl.num_programs(0) - 1)
    def _fin():
        out_ref[...] = jnp.full((1, 1), acc_ref[0], dtype=jnp.float32)


def _make_sc_sumsq(n_tokens):
    rows_per_w = n_tokens // NW
    nchunk = rows_per_w // SCROWS
    mesh = plsc.VectorSubcoreMesh(core_axis_name="c", subcore_axis_name="s")

    @functools.partial(
        pl.kernel, mesh=mesh,
        out_type=jax.ShapeDtypeStruct((NW, 16), jnp.float32),
        scratch_types=[
            pltpu.VMEM((SCROWS, DIM), jnp.float32),
            pltpu.VMEM((SCROWS, DIM), jnp.float32),
            pltpu.VMEM((16,), jnp.float32),
            pltpu.SemaphoreType.DMA,
            pltpu.SemaphoreType.DMA,
        ],
    )
    def sc_sumsq(x_hbm, out_hbm, buf0, buf1, acc_v, sem0, sem1):
        wid = lax.axis_index("s") * 2 + lax.axis_index("c")
        base = wid * rows_per_w
        acc_v[...] = jnp.zeros((16,), jnp.float32)
        bufs = (buf0, buf1)
        sems = (sem0, sem1)
        cps = [None, None]
        cps[0] = pltpu.async_copy(
            x_hbm.at[pl.ds(base, SCROWS)], buf0, sem0)
        for j in range(nchunk):
            nxt = (j + 1) % 2
            cur = j % 2
            if j + 1 < nchunk:
                cps[nxt] = pltpu.async_copy(
                    x_hbm.at[pl.ds(base + (j + 1) * SCROWS, SCROWS)],
                    bufs[nxt], sems[nxt])
            cps[cur].wait()
            buf = bufs[cur]

            @pl.loop(0, SCROWS)
            def _row(r):
                parts = [jnp.zeros((16,), jnp.float32) for _ in range(4)]
                for l in range(DIM // 16):
                    v = buf[r, pl.ds(l * 16, 16)]
                    parts[l % 4] = parts[l % 4] + v * v
                acc_v[...] = acc_v[...] + ((parts[0] + parts[1]) +
                                           (parts[2] + parts[3]))

        pltpu.sync_copy(acc_v, out_hbm.at[wid])

    return sc_sumsq


def kernel(x, W, b, to_output, mask):
    del mask
    n_tokens = x.shape[0]

    w4 = W.reshape(NCB, CB, NCB, DPC)
    t4 = to_output.reshape(NCB, CB, NCB, DPC)
    diag = jnp.arange(NCB)
    wblk = w4[diag, :, diag, :]
    tblk = t4[diag, :, diag, :]
    a = jnp.concatenate(
        [jnp.transpose(wblk, (0, 2, 1)).reshape(NCB * DPC, CB),
         -2.0 * jnp.transpose(tblk, (0, 2, 1)).reshape(NCB * DPC, CB)],
        axis=1).astype(jnp.bfloat16)
    b2 = b.reshape(NCB, CB)

    grid = n_tokens // TBLK
    num = pl.pallas_call(
        _vq_kernel,
        grid=(grid,),
        in_specs=[
            pl.BlockSpec((TBLK, DIM), lambda i: (i, 0)),
            pl.BlockSpec((NCB * DPC, 2 * CB), lambda i: (0, 0)),
            pl.BlockSpec((NCB, CB), lambda i: (0, 0)),
        ],
        out_specs=pl.BlockSpec((1, 1), lambda i: (0, 0)),
        out_shape=jax.ShapeDtypeStruct((1, 1), jnp.float32),
        scratch_shapes=[pltpu.VMEM((NCB, CB), jnp.float32),
                        pltpu.SMEM((1,), jnp.float32)],
    )(x, a, b2)

    sxp = _make_sc_sumsq(n_tokens)(x)
    sx = jnp.sum(sxp)
    return (num.reshape(()) + sx) / (sx + 1e-20)
